# Initial kernel scaffold; baseline (speedup 1.0000x reference)
import jax
import jax.numpy as jnp
from jax.experimental import pallas as pl


def kernel(scores, label, seqlen):
    raise NotImplementedError("write your pallas kernel here")


# speedup vs baseline: 8.4873x; 8.4873x over previous
import functools
import jax
import jax.numpy as jnp
from jax import lax
from jax.experimental import pallas as pl
from jax.experimental.pallas import tpu as pltpu

K = 64


def _loss_body(scores_ref, label_ref, seqlen_ref, out_ref):
    B, N = scores_ref.shape
    x = scores_ref[...]
    seql = seqlen_ref[...]
    label = label_ref[...]

    pos = lax.broadcasted_iota(jnp.int32, (B, N), 1)
    valid = pos < seql

    ui = lax.bitcast_convert_type(x, jnp.uint32)
    neg = ui >= jnp.uint32(0x80000000)
    u = jnp.where(neg, ~ui, ui | jnp.uint32(0x80000000))
    u = jnp.where(valid, u, jnp.uint32(0))

    k_i = jnp.minimum(seql, K)

    t = jnp.zeros((B, 1), jnp.uint32)
    for b in range(31, -1, -1):
        cand = t | jnp.uint32(1 << b)
        cnt = jnp.sum((u >= cand).astype(jnp.int32), axis=1, keepdims=True)
        t = jnp.where(cnt >= k_i, cand, t)

    gt = u > t
    cnt_gt = jnp.sum(gt.astype(jnp.int32), axis=1, keepdims=True)
    sum_gt = jnp.sum(jnp.where(gt, x, 0.0), axis=1, keepdims=True)
    tv = lax.bitcast_convert_type(
        jnp.where(t >= jnp.uint32(0x80000000), t ^ jnp.uint32(0x80000000), ~t),
        jnp.float32,
    )
    kf = k_i.astype(jnp.float32)
    total = sum_gt + (kf - cnt_gt.astype(jnp.float32)) * tv
    mean = total / kf

    p = jax.nn.sigmoid(mean)
    eps = 1e-7
    p = jnp.clip(p, eps, 1.0 - eps)
    bce = -(label * jnp.log(p) + (1.0 - label) * jnp.log(1.0 - p))
    out_ref[0, 0] = jnp.mean(bce)


@jax.jit
def kernel(scores, label, seqlen):
    B, N = scores.shape
    out = pl.pallas_call(
        _loss_body,
        out_shape=jax.ShapeDtypeStruct((1, 1), jnp.float32),
        out_specs=pl.BlockSpec(memory_space=pltpu.SMEM),
    )(scores, label.reshape(B, 1), seqlen.astype(jnp.int32).reshape(B, 1))
    return out[0, 0]
